# baseline (speedup 1.0000x reference)
import functools

import jax
import jax.numpy as jnp
from jax import lax
from jax.experimental import pallas as pl
from jax.experimental.pallas import tpu as pltpu
from jax.experimental.pallas import tpu_sc as plsc

B, N, T = 1024, 26, 20
M = B * N * T
D = 16
V = 100000
NC, NS = 2, 16
NW = NC * NS
CHUNK = M // NW
SUB = 1040
N_ITERS = CHUNK // SUB

_mesh = plsc.VectorSubcoreMesh(core_axis_name="c", subcore_axis_name="s")


@functools.partial(
    pl.kernel,
    mesh=_mesh,
    compiler_params=pltpu.CompilerParams(use_tc_tiling_on_sc=False),
    out_type=jax.ShapeDtypeStruct((M, 3 * D), jnp.float32),
    scratch_types=[
        [[pltpu.VMEM((SUB,), jnp.int32)] * 3] * 2,
        [[pltpu.VMEM((SUB, D), jnp.float32)] * 3] * 2,
        [pltpu.SemaphoreType.DMA] * 2,
        [pltpu.SemaphoreType.DMA] * 2,
        [pltpu.SemaphoreType.DMA] * 2,
    ],
)
def _embed(xt, table, out, idx_v, rows_v, sem_i, sem_g, sem_w):
    wid = lax.axis_index("s") * NC + lax.axis_index("c")
    base = wid * CHUNK

    idx_cps = {}
    gathers = {}
    writes = {}

    def fetch_idx(j, s):
        p0 = base + j * SUB
        for i in range(3):
            idx_cps[(j, i)] = pltpu.async_copy(
                xt.at[i, pl.ds(p0, SUB)], idx_v[s][i], sem_i[s]
            )

    def start_gathers(j, s):
        for i in range(3):
            idx_cps[(j, i)].wait()
        for i in range(3):
            gathers[(j, i)] = pltpu.async_copy(
                table.at[idx_v[s][i]], rows_v[s][i], sem_g[s]
            )

    def write_out(j, s):
        p0 = base + j * SUB
        for i in range(3):
            gathers[(j, i)].wait()
        for i in range(3):
            writes[(j, i)] = pltpu.async_copy(
                rows_v[s][i], out.at[pl.ds(p0, SUB), pl.ds(i * D, D)], sem_w[s]
            )

    fetch_idx(0, 0)
    for j in range(N_ITERS):
        s = j % 2
        if j >= 2:
            for i in range(3):
                writes[(j - 2, i)].wait()
        start_gathers(j, s)
        if j >= 1:
            write_out(j - 1, 1 - s)
        if j + 1 < N_ITERS:
            fetch_idx(j + 1, 1 - s)
    write_out(N_ITERS - 1, (N_ITERS - 1) % 2)
    for j in (N_ITERS - 2, N_ITERS - 1):
        for i in range(3):
            writes[(j, i)].wait()


def kernel(x, W0, W1, W2):
    table = jnp.concatenate([W0, W1, W2], axis=0)
    xt = (x + jnp.arange(3, dtype=jnp.int32) * V).reshape(M, 3).T
    out = _embed(xt, table)
    return out.reshape(B, N, T, 3 * D)
